# Initial kernel scaffold; baseline (speedup 1.0000x reference)
import functools

import jax
import jax.numpy as jnp
from jax import lax
from jax.experimental import pallas as pl
from jax.experimental.pallas import tpu as pltpu
from jax.experimental.pallas import tpu_sc as plsc

NC = 2
NS = 16
LANES = 16

CH = 128


def _agg_body(B, N, E, xf, src, dst, val, out, acc, src_v, gidx_v, dst_v,
              val_v, rows_v, src_t, gidx_t, dst_t, val_t, rows_t, zbuf, sem):
    c = lax.axis_index("c")
    s = lax.axis_index("s")

    e_per_tile = E // NS
    n_chunks = e_per_tile // CH
    tail = e_per_tile - n_chunks * CH
    n_per_tile = N // NS

    def zrow(r, _):
        for j in range(8):
            zbuf[r, pl.ds(j * 16, 16)] = jnp.zeros((16,), jnp.float32)
        return 0
    lax.fori_loop(0, n_per_tile, zrow, 0)

    def process(base, cnt, boff, sv, gv, dv, vv, rv):
        pltpu.sync_copy(src.at[pl.ds(base, cnt)], sv)
        pltpu.sync_copy(dst.at[pl.ds(base, cnt)], dv)
        pltpu.sync_copy(val.at[pl.ds(base, cnt)], vv)
        for j in range(cnt // 16):
            s16 = sv[pl.ds(j * 16, 16)]
            gv[pl.ds(j * 16, 16)] = s16 * 2 + boff
        pltpu.async_copy(xf.at[gv], rv, sem).wait()

        def erow(e, _):
            v = vv[e]
            for j in range(8):
                rv[e, pl.ds(j * 16, 16)] = rv[e, pl.ds(j * 16, 16)] * v
            return 0
        lax.fori_loop(0, cnt, erow, 0)

        pltpu.sync_copy(rv, acc.at[dv], add=True)

    def batch_body(b, _):
        pltpu.sync_copy(zbuf, acc.at[pl.ds(s * n_per_tile, n_per_tile)])
        plsc.subcore_barrier()

        boff = 2 * b * N + c

        def chunk_body(i, _):
            process(s * e_per_tile + i * CH, CH, boff,
                    src_v, gidx_v, dst_v, val_v, rows_v)
            return 0
        lax.fori_loop(0, n_chunks, chunk_body, 0)
        if tail:
            process(s * e_per_tile + n_chunks * CH, tail, boff,
                    src_t, gidx_t, dst_t, val_t, rows_t)

        plsc.subcore_barrier()
        off = (c * B + b) * N + s * n_per_tile
        pltpu.sync_copy(acc.at[pl.ds(s * n_per_tile, n_per_tile)],
                        out.at[pl.ds(off, n_per_tile)])
        plsc.subcore_barrier()
        return 0

    lax.fori_loop(0, B, batch_body, 0)


def _sc_aggregate(xf, src, dst, val, B, N, E):
    e_per_tile = E // NS
    tail = e_per_tile - (e_per_tile // CH) * CH
    n_per_tile = N // NS
    mesh = plsc.VectorSubcoreMesh(core_axis_name="c", subcore_axis_name="s")
    kern = pl.kernel(
        functools.partial(_agg_body, B, N, E),
        out_type=jax.ShapeDtypeStruct((NC * B * N, 128), jnp.float32),
        mesh=mesh,
        scratch_types=[
            pltpu.VMEM_SHARED((N, 128), jnp.float32),
            pltpu.VMEM((CH,), jnp.int32),
            pltpu.VMEM((CH,), jnp.int32),
            pltpu.VMEM((CH,), jnp.int32),
            pltpu.VMEM((CH,), jnp.float32),
            pltpu.VMEM((CH, 128), jnp.float32),
            pltpu.VMEM((max(tail, 16),), jnp.int32),
            pltpu.VMEM((max(tail, 16),), jnp.int32),
            pltpu.VMEM((max(tail, 16),), jnp.int32),
            pltpu.VMEM((max(tail, 16),), jnp.float32),
            pltpu.VMEM((max(tail, 16), 128), jnp.float32),
            pltpu.VMEM((n_per_tile, 128), jnp.float32),
            pltpu.SemaphoreType.DMA,
        ],
    )
    return kern(xf, src, dst, val)


def _matmul_body(aref, wref, oref):
    a = aref[...]
    w = wref[...]
    r = (jnp.dot(a[0], w[0], preferred_element_type=jnp.float32)
         + jnp.dot(a[1], w[1], preferred_element_type=jnp.float32))
    oref[...] = jnp.maximum(r, 0.0)


def _tc_matmul(agg2, W2, BN=1000):
    M = agg2.shape[1]
    grid = (M // BN,)
    return pl.pallas_call(
        _matmul_body,
        grid=grid,
        in_specs=[
            pl.BlockSpec((2, BN, 128), lambda i: (0, i, 0)),
            pl.BlockSpec((2, 128, 512), lambda i: (0, 0, 0)),
        ],
        out_specs=pl.BlockSpec((BN, 512), lambda i: (i, 0)),
        out_shape=jax.ShapeDtypeStruct((M, 512), jnp.float32),
    )(agg2, W2)


def kernel(x, nl_ind, nl_value, W):
    B, N, D = x.shape
    E = nl_ind.shape[0]
    xf = x.reshape(B * N * 2, 128)
    src = nl_ind[:, 1].astype(jnp.int32)
    dst = nl_ind[:, 0].astype(jnp.int32)
    val = nl_value.astype(jnp.float32)

    agg = _sc_aggregate(xf, src, dst, val, B, N, E)
    agg2 = agg.reshape(2, B * N, 128)
    W2 = W.reshape(2, 128, 512)
    out = _tc_matmul(agg2, W2)
    return out.reshape(B, N, 512)


# speedup vs baseline: 7.8523x; 7.8523x over previous
import functools

import jax
import jax.numpy as jnp
from jax import lax
from jax.experimental import pallas as pl
from jax.experimental.pallas import tpu as pltpu
from jax.experimental.pallas import tpu_sc as plsc

NC = 2
NS = 16
LANES = 16

CH = 128
ZROWS = 16


def _agg_body(B, N, E, xf, src, dst, val, out, acc, src_v, gidx_v, dst_v,
              val_v, rows_v, src_t, gidx_t, dst_t, val_t, rows_t, zbuf, sem):
    c = lax.axis_index("c")
    s = lax.axis_index("s")

    e_per_tile = E // NS
    n_chunks = e_per_tile // CH
    tail = e_per_tile - n_chunks * CH
    n_base = (N // NS) & ~7
    n_rem = N - NS * n_base

    for r in range(ZROWS):
        for j in range(8):
            zbuf[r, pl.ds(j * 16, 16)] = jnp.zeros((16,), jnp.float32)

    def process(base, cnt, boff, sv, gv, dv, vv, rv):
        pltpu.sync_copy(src.at[pl.ds(base, cnt)], sv)
        pltpu.sync_copy(dst.at[pl.ds(base, cnt)], dv)
        pltpu.sync_copy(val.at[pl.ds(base, cnt)], vv)
        for j in range(cnt // 16):
            s16 = sv[pl.ds(j * 16, 16)]
            gv[pl.ds(j * 16, 16)] = s16 * 2 + boff
        pltpu.async_copy(xf.at[gv], rv, sem).wait()

        def egroup(g, _):
            vvec = vv[pl.ds(g * 16, 16)]
            for l in range(16):
                v = vvec[l]
                e = g * 16 + l
                for j in range(8):
                    rv[e, pl.ds(j * 16, 16)] = rv[e, pl.ds(j * 16, 16)] * v
            return 0
        lax.fori_loop(0, cnt // 16, egroup, 0)

        pltpu.sync_copy(rv, acc.at[dv], add=True)

    def batch_body(b, _):
        def zcopy(z, _):
            pltpu.sync_copy(zbuf, acc.at[pl.ds(s * n_base + z * ZROWS, ZROWS)])
            return 0
        lax.fori_loop(0, n_base // ZROWS, zcopy, 0)
        if n_rem:
            @pl.when(s == 0)
            def _():
                pltpu.sync_copy(zbuf.at[pl.ds(0, n_rem)],
                                acc.at[pl.ds(NS * n_base, n_rem)])
        plsc.subcore_barrier()

        boff = 2 * b * N + c

        def chunk_body(i, _):
            process(s * e_per_tile + i * CH, CH, boff,
                    src_v, gidx_v, dst_v, val_v, rows_v)
            return 0
        lax.fori_loop(0, n_chunks, chunk_body, 0)
        if tail:
            process(s * e_per_tile + n_chunks * CH, tail, boff,
                    src_t, gidx_t, dst_t, val_t, rows_t)

        plsc.subcore_barrier()
        off = (c * B + b) * N
        pltpu.sync_copy(acc.at[pl.ds(s * n_base, n_base)],
                        out.at[pl.ds(off + s * n_base, n_base)])
        if n_rem:
            @pl.when(s == 0)
            def _():
                pltpu.sync_copy(acc.at[pl.ds(NS * n_base, n_rem)],
                                out.at[pl.ds(off + NS * n_base, n_rem)])
        plsc.subcore_barrier()
        return 0

    lax.fori_loop(0, B, batch_body, 0)


def _sc_aggregate(xf, src, dst, val, B, N, E):
    e_per_tile = E // NS
    tail = e_per_tile - (e_per_tile // CH) * CH
    n_base = (N // NS) & ~7
    mesh = plsc.VectorSubcoreMesh(core_axis_name="c", subcore_axis_name="s",
                                  num_cores=NC, num_subcores=NS)
    kern = pl.kernel(
        functools.partial(_agg_body, B, N, E),
        out_type=jax.ShapeDtypeStruct((NC * B * N, 128), jnp.float32),
        mesh=mesh,
        scratch_types=[
            pltpu.VMEM_SHARED((N, 128), jnp.float32),
            pltpu.VMEM((CH,), jnp.int32),
            pltpu.VMEM((CH,), jnp.int32),
            pltpu.VMEM((CH,), jnp.int32),
            pltpu.VMEM((CH,), jnp.float32),
            pltpu.VMEM((CH, 128), jnp.float32),
            pltpu.VMEM((max(tail, 16),), jnp.int32),
            pltpu.VMEM((max(tail, 16),), jnp.int32),
            pltpu.VMEM((max(tail, 16),), jnp.int32),
            pltpu.VMEM((max(tail, 16),), jnp.float32),
            pltpu.VMEM((max(tail, 16), 128), jnp.float32),
            pltpu.VMEM((ZROWS, 128), jnp.float32),
            pltpu.SemaphoreType.DMA,
        ],
    )
    return kern(xf, src, dst, val)


def _matmul_body(aref, wref, oref):
    a = aref[...]
    w = wref[...]
    r = (jnp.dot(a[0], w[0], preferred_element_type=jnp.float32)
         + jnp.dot(a[1], w[1], preferred_element_type=jnp.float32))
    oref[...] = jnp.maximum(r, 0.0)


def _tc_matmul(agg2, W2, BN=1000):
    M = agg2.shape[1]
    grid = (M // BN,)
    return pl.pallas_call(
        _matmul_body,
        grid=grid,
        in_specs=[
            pl.BlockSpec((2, BN, 128), lambda i: (0, i, 0)),
            pl.BlockSpec((2, 128, 512), lambda i: (0, 0, 0)),
        ],
        out_specs=pl.BlockSpec((BN, 512), lambda i: (i, 0)),
        out_shape=jax.ShapeDtypeStruct((M, 512), jnp.float32),
    )(agg2, W2)


def kernel(x, nl_ind, nl_value, W):
    B, N, D = x.shape
    E = nl_ind.shape[0]
    xf = x.reshape(B * N * 2, 128)
    src = nl_ind[:, 1].astype(jnp.int32)
    dst = nl_ind[:, 0].astype(jnp.int32)
    val = nl_value.astype(jnp.float32)

    agg = _sc_aggregate(xf, src, dst, val, B, N, E)
    agg2 = agg.reshape(2, B * N, 128)
    W2 = W.reshape(2, 128, 512)
    out = _tc_matmul(agg2, W2)
    return out.reshape(B, N, 512)


# speedup vs baseline: 7.9867x; 1.0171x over previous
import functools

import jax
import jax.numpy as jnp
from jax import lax
from jax.experimental import pallas as pl
from jax.experimental.pallas import tpu as pltpu
from jax.experimental.pallas import tpu_sc as plsc

NC = 2
NS = 16

CH = 128
ZROWS = 16


def _scale_rows(ebuf, rbuf):
    def egroup(g, _):
        vbits = ebuf[2, pl.ds(g * 16, 16)]
        vvec = lax.bitcast_convert_type(vbits, jnp.float32)
        for l in range(16):
            v = vvec[l]
            e = g * 16 + l
            for j in range(8):
                rbuf[e, pl.ds(j * 16, 16)] = rbuf[e, pl.ds(j * 16, 16)] * v
        return 0
    lax.fori_loop(0, CH // 16, egroup, 0)


def _make_gidx(ebuf, gx, boff):
    for j in range(CH // 16):
        gx[pl.ds(j * 16, 16)] = ebuf[0, pl.ds(j * 16, 16)] + boff


def _copy_dst(ebuf, dbuf):
    for j in range(CH // 16):
        dbuf[pl.ds(j * 16, 16)] = ebuf[1, pl.ds(j * 16, 16)]


def _agg_body(B, N, NCH, xf, edata, out, acc,
              e0, e1, gx0, gx1, d0, d1, r0, r1, zbuf,
              es0, es1, gs0, gs1, ss0, ss1):
    c = lax.axis_index("c")
    s = lax.axis_index("s")

    n_pairs = NCH // 2
    n_base = (N // NS) & ~7
    n_rem = N - NS * n_base

    for r in range(ZROWS):
        for j in range(8):
            zbuf[r, pl.ds(j * 16, 16)] = jnp.zeros((16,), jnp.float32)

    cbase = s * NCH

    def batch_body(b, _):
        def zcopy(z, _):
            pltpu.sync_copy(zbuf, acc.at[pl.ds(s * n_base + z * ZROWS, ZROWS)])
            return 0
        lax.fori_loop(0, n_base // ZROWS, zcopy, 0)
        if n_rem:
            @pl.when(s == 0)
            def _():
                pltpu.sync_copy(zbuf.at[pl.ds(0, n_rem)],
                                acc.at[pl.ds(NS * n_base, n_rem)])
        plsc.subcore_barrier()

        boff = 2 * b * N + c

        pltpu.async_copy(edata.at[cbase], e0, es0)
        pltpu.async_copy(edata.at[cbase + 1], e1, es1)
        pltpu.make_async_copy(edata.at[cbase], e0, es0).wait()
        _make_gidx(e0, gx0, boff)
        pltpu.async_copy(xf.at[gx0], r0, gs0)

        def pair_body(p, _):
            a = cbase + 2 * p
            bch = a + 1
            pltpu.make_async_copy(edata.at[bch], e1, es1).wait()
            _make_gidx(e1, gx1, boff)

            @pl.when(p > 0)
            def _():
                pltpu.make_async_copy(r1, acc.at[d1], ss1).wait()
            pltpu.async_copy(xf.at[gx1], r1, gs1)

            pltpu.make_async_copy(xf.at[gx0], r0, gs0).wait()
            _scale_rows(e0, r0)
            _copy_dst(e0, d0)
            pltpu.async_copy(r0, acc.at[d0], ss0, add=True)

            @pl.when(p < n_pairs - 1)
            def _():
                pltpu.async_copy(edata.at[a + 2], e0, es0)

            pltpu.make_async_copy(xf.at[gx1], r1, gs1).wait()
            _scale_rows(e1, r1)
            _copy_dst(e1, d1)
            pltpu.async_copy(r1, acc.at[d1], ss1, add=True)

            @pl.when(p < n_pairs - 1)
            def _():
                pltpu.async_copy(edata.at[a + 3], e1, es1)
                pltpu.make_async_copy(edata.at[a + 2], e0, es0).wait()
                _make_gidx(e0, gx0, boff)
                pltpu.make_async_copy(r0, acc.at[d0], ss0).wait()
                pltpu.async_copy(xf.at[gx0], r0, gs0)
            return 0

        lax.fori_loop(0, n_pairs, pair_body, 0)
        pltpu.make_async_copy(r0, acc.at[d0], ss0).wait()
        pltpu.make_async_copy(r1, acc.at[d1], ss1).wait()

        plsc.subcore_barrier()
        off = (c * B + b) * N
        pltpu.sync_copy(acc.at[pl.ds(s * n_base, n_base)],
                        out.at[pl.ds(off + s * n_base, n_base)])
        if n_rem:
            @pl.when(s == 0)
            def _():
                pltpu.sync_copy(acc.at[pl.ds(NS * n_base, n_rem)],
                                out.at[pl.ds(off + NS * n_base, n_rem)])
        plsc.subcore_barrier()
        return 0

    lax.fori_loop(0, B, batch_body, 0)


def _sc_aggregate(xf, edata, B, N, NCH):
    mesh = plsc.VectorSubcoreMesh(core_axis_name="c", subcore_axis_name="s",
                                  num_cores=NC, num_subcores=NS)
    kern = pl.kernel(
        functools.partial(_agg_body, B, N, NCH),
        out_type=jax.ShapeDtypeStruct((NC * B * N, 128), jnp.float32),
        mesh=mesh,
        scratch_types=[
            pltpu.VMEM_SHARED((N, 128), jnp.float32),
            pltpu.VMEM((3, CH), jnp.int32),
            pltpu.VMEM((3, CH), jnp.int32),
            pltpu.VMEM((CH,), jnp.int32),
            pltpu.VMEM((CH,), jnp.int32),
            pltpu.VMEM((CH,), jnp.int32),
            pltpu.VMEM((CH,), jnp.int32),
            pltpu.VMEM((CH, 128), jnp.float32),
            pltpu.VMEM((CH, 128), jnp.float32),
            pltpu.VMEM((ZROWS, 128), jnp.float32),
            pltpu.SemaphoreType.DMA,
            pltpu.SemaphoreType.DMA,
            pltpu.SemaphoreType.DMA,
            pltpu.SemaphoreType.DMA,
            pltpu.SemaphoreType.DMA,
            pltpu.SemaphoreType.DMA,
        ],
    )
    return kern(xf, edata)


def _matmul_body(aref, wref, oref):
    a = aref[...]
    w = wref[...]
    r = (jnp.dot(a[0], w[0], preferred_element_type=jnp.float32)
         + jnp.dot(a[1], w[1], preferred_element_type=jnp.float32))
    oref[...] = jnp.maximum(r, 0.0)


def _tc_matmul(agg2, W2, BN=1000):
    M = agg2.shape[1]
    grid = (M // BN,)
    return pl.pallas_call(
        _matmul_body,
        grid=grid,
        in_specs=[
            pl.BlockSpec((2, BN, 128), lambda i: (0, i, 0)),
            pl.BlockSpec((2, 128, 512), lambda i: (0, 0, 0)),
        ],
        out_specs=pl.BlockSpec((BN, 512), lambda i: (i, 0)),
        out_shape=jax.ShapeDtypeStruct((M, 512), jnp.float32),
    )(agg2, W2)


def kernel(x, nl_ind, nl_value, W):
    B, N, D = x.shape
    E = nl_ind.shape[0]
    xf = x.reshape(B * N * 2, 128)

    gran = 2 * CH * NS
    E_pad = ((E + gran - 1) // gran) * gran
    pad = E_pad - E
    src2 = jnp.pad(nl_ind[:, 1].astype(jnp.int32) * 2, (0, pad))
    dstp = jnp.pad(nl_ind[:, 0].astype(jnp.int32), (0, pad))
    valp = jnp.pad(lax.bitcast_convert_type(nl_value.astype(jnp.float32),
                                            jnp.int32), (0, pad))
    edata = jnp.stack([src2.reshape(-1, CH), dstp.reshape(-1, CH),
                       valp.reshape(-1, CH)], axis=1)
    NCH = E_pad // (CH * NS)

    agg = _sc_aggregate(xf, edata, B, N, NCH)
    agg2 = agg.reshape(2, B * N, 128)
    W2 = W.reshape(2, 128, 512)
    out = _tc_matmul(agg2, W2)
    return out.reshape(B, N, 512)
